# baseline (speedup 1.0000x reference)
import functools

import jax
import jax.numpy as jnp
import numpy as np
from jax import lax
from jax.experimental import pallas as pl
from jax.experimental.pallas import tpu as pltpu
from jax.experimental.pallas import tpu_sc as plsc

_PL = 48
_NC = 16
_DIM = 3


@functools.lru_cache(maxsize=None)
def _make_knn(B, N, M, K, Mb):
    grid = (B, M // Mb)

    def body(npts_ref, ptst_ref, idx_ref):
        q = npts_ref[0]
        p = ptst_ref[0]
        d2 = (q[:, 0:1] - p[0:1, :]) ** 2
        d2 = d2 + (q[:, 1:2] - p[1:2, :]) ** 2
        d2 = d2 + (q[:, 2:3] - p[2:3, :]) ** 2
        iota = jax.lax.broadcasted_iota(jnp.int32, (Mb, N), 1)
        work = d2
        cols = []
        for _ in range(K):
            ik = jnp.argmin(work, axis=1).astype(jnp.int32)[:, None]
            cols.append(ik)
            work = jnp.where(iota == ik, jnp.float32(jnp.inf), work)
        idx_ref[0] = jnp.concatenate(cols, axis=1)

    return pl.pallas_call(
        body,
        grid=grid,
        in_specs=[
            pl.BlockSpec((1, Mb, 3), lambda b, i: (b, i, 0)),
            pl.BlockSpec((1, 3, N), lambda b, i: (b, 0, 0)),
        ],
        out_specs=pl.BlockSpec((1, Mb, K), lambda b, i: (b, i, 0)),
        out_shape=jax.ShapeDtypeStruct((B, M, K), jnp.int32),
    )


def _knn_idx(pts_src, pts_query, K):
    B, N, _ = pts_src.shape
    M = pts_query.shape[1]
    Mb = min(M, 256)
    pts_t = pts_src.transpose(0, 2, 1)
    return _make_knn(B, N, M, K, Mb)(pts_query, pts_t)


def _gather(arr, idx):
    return jax.vmap(lambda a, i: a[i])(arr, idx)


_SC_CHUNK = 128


@functools.lru_cache(maxsize=None)
def _make_sc_gather(G, R):
    n_w = 32
    per_w = G // n_w
    n_iter = per_w // _SC_CHUNK
    mesh = plsc.VectorSubcoreMesh(core_axis_name="c", subcore_axis_name="s")

    @functools.partial(
        pl.kernel,
        mesh=mesh,
        out_type=jax.ShapeDtypeStruct((G, R), jnp.float32),
        scratch_types=[
            pltpu.VMEM((_SC_CHUNK,), jnp.int32),
            pltpu.VMEM((_SC_CHUNK, R), jnp.float32),
            pltpu.SemaphoreType.DMA,
        ],
    )
    def k(table_hbm, gid_hbm, out_hbm, idx_v, rows_v, sem):
        wid = lax.axis_index("s") * 2 + lax.axis_index("c")
        base = wid * per_w

        def one(i, carry):
            off = base + i * _SC_CHUNK
            pltpu.sync_copy(gid_hbm.at[pl.ds(off, _SC_CHUNK)], idx_v)
            pltpu.async_copy(table_hbm.at[idx_v], rows_v, sem).wait()
            pltpu.sync_copy(rows_v, out_hbm.at[pl.ds(off, _SC_CHUNK)])
            return carry

        lax.fori_loop(0, n_iter, one, 0)

    return k


def _sc_gather_pair(x, pts, idx):
    B, N, Cin = x.shape
    M, K = idx.shape[1], idx.shape[2]
    R = ((Cin + 3 + 127) // 128) * 128
    table = jnp.concatenate(
        [x.reshape(B * N, Cin), pts.reshape(B * N, 3),
         jnp.zeros((B * N, R - Cin - 3), jnp.float32)], axis=1)
    gid = (idx + (jnp.arange(B, dtype=jnp.int32) * N)[:, None, None]).reshape(-1)
    rows = _make_sc_gather(B * M * K, R)(table, gid)
    feats = rows[:, :Cin].reshape(B, M, K, Cin)
    ptsg = rows[:, Cin:Cin + 3].reshape(B, M, K, 3)
    return feats, ptsg


def _ptconv(p, x, pts, K, next_pts, use_sc=True):
    B, N, Cin = x.shape
    if isinstance(next_pts, int):
        if next_pts == N:
            npts = pts
        else:
            stride = N // next_pts
            npts = pts[:, ::stride][:, :next_pts]
    else:
        npts = next_pts
    M = npts.shape[1]
    idx = _knn_idx(pts, npts, K)
    if use_sc and (B * M * K) % (32 * _SC_CHUNK) == 0:
        feats, ptsg = _sc_gather_pair(x, pts, idx)
    else:
        feats = _gather(x, idx)
        ptsg = _gather(pts, idx)
    rel = ptsg - npts[:, :, None, :]
    maxi = jnp.sqrt(jnp.max(jnp.sum(jax.lax.stop_gradient(rel) ** 2, axis=-1), axis=-1))
    maxi = jnp.where(maxi == 0.0, 1.0, maxi)
    rel = rel / maxi[:, :, None, None]
    dists = (rel[:, :, :, None, :] - p["centers"]).reshape(B, M, K, -1)
    h = jax.nn.relu(dists @ p["l1w"] + p["l1b"])
    h = jax.nn.relu(h @ p["l2w"] + p["l2b"])
    h = jax.nn.relu(h @ p["l3w"] + p["l3b"])
    fs = jnp.einsum('bmkc,bmkn->bmcn', feats, h).reshape(B, M, -1)
    out = fs @ p["weight"].reshape(-1, p["weight"].shape[2])
    out = out / K
    return out, npts


def _bn(bp, x):
    mean = jnp.mean(x, axis=(0, 1))
    var = jnp.var(x, axis=(0, 1))
    return bp["gamma"] * (x - mean) / jnp.sqrt(var + 1e-5) + bp["beta"]


def _fc_kernel(x_ref, w_ref, b_ref, o_ref):
    o_ref[...] = jnp.dot(x_ref[...], w_ref[...],
                         preferred_element_type=jnp.float32) + b_ref[...]


def _fc_pallas(x2d, w, b):
    R, C = x2d.shape
    O = w.shape[1]
    return pl.pallas_call(
        _fc_kernel,
        out_shape=jax.ShapeDtypeStruct((R, O), jnp.float32),
    )(x2d, w, b[None, :])


def kernel(x, input_pts, params):
    r = jax.nn.relu
    x1, pts1 = _ptconv(params["cv1"], x, input_pts, 16, 2048)
    x1 = r(_bn(params["bn_cv1"], x1))
    x2, pts2 = _ptconv(params["cv2"], x1, pts1, 16, 1024)
    x2 = r(_bn(params["bn_cv2"], x2))
    x3, pts3 = _ptconv(params["cv3"], x2, pts2, 16, 256)
    x3 = r(_bn(params["bn_cv3"], x3))
    x4, pts4 = _ptconv(params["cv4"], x3, pts3, 8, 64)
    x4 = r(_bn(params["bn_cv4"], x4))
    x5, pts5 = _ptconv(params["cv5"], x4, pts4, 8, 16)
    x5 = r(_bn(params["bn_cv5"], x5))
    x6, pts6 = _ptconv(params["cv6"], x5, pts5, 4, 8)
    x6 = r(_bn(params["bn_cv6"], x6))
    x5d, _ = _ptconv(params["cv5d"], x6, pts6, 4, pts5)
    x5d = r(_bn(params["bn_cv5d"], x5d))
    x5d = jnp.concatenate([x5d, x5], axis=2)
    x4d, _ = _ptconv(params["cv4d"], x5d, pts5, 4, pts4)
    x4d = r(_bn(params["bn_cv4d"], x4d))
    x4d = jnp.concatenate([x4d, x4], axis=2)
    x3d, _ = _ptconv(params["cv3d"], x4d, pts4, 4, pts3)
    x3d = r(_bn(params["bn_cv3d"], x3d))
    x3d = jnp.concatenate([x3d, x3], axis=2)
    x2d, _ = _ptconv(params["cv2d"], x3d, pts3, 8, pts2)
    x2d = r(_bn(params["bn_cv2d"], x2d))
    x2d = jnp.concatenate([x2d, x2], axis=2)
    x1d, _ = _ptconv(params["cv1d"], x2d, pts2, 8, pts1)
    x1d = r(_bn(params["bn_cv1d"], x1d))
    x1d = jnp.concatenate([x1d, x1], axis=2)
    x0d, _ = _ptconv(params["cv0d"], x1d, pts1, 8, input_pts)
    x0d = r(_bn(params["bn_cv0d"], x0d))
    B, M, C = x0d.shape
    xout = _fc_pallas(x0d.reshape(-1, C), params["fcout_w"], params["fcout_b"])
    return xout.reshape(B, -1, xout.shape[1])


# speedup vs baseline: 1.0080x; 1.0080x over previous
import functools

import jax
import jax.numpy as jnp
import numpy as np
from jax import lax
from jax.experimental import pallas as pl
from jax.experimental.pallas import tpu as pltpu
from jax.experimental.pallas import tpu_sc as plsc

_PL = 48
_NC = 16
_DIM = 3


@functools.lru_cache(maxsize=None)
def _make_knn(B, N, M, K, Mb):
    grid = (B, M // Mb)

    def body(npts_ref, ptst_ref, idx_ref):
        q = npts_ref[0]
        p = ptst_ref[0]
        d2 = (q[:, 0:1] - p[0:1, :]) ** 2
        d2 = d2 + (q[:, 1:2] - p[1:2, :]) ** 2
        d2 = d2 + (q[:, 2:3] - p[2:3, :]) ** 2
        iota = jax.lax.broadcasted_iota(jnp.int32, (Mb, N), 1)
        work = d2
        cols = []
        for _ in range(K):
            ik = jnp.argmin(work, axis=1).astype(jnp.int32)[:, None]
            cols.append(ik)
            work = jnp.where(iota == ik, jnp.float32(jnp.inf), work)
        idx_ref[0] = jnp.concatenate(cols, axis=1)

    return pl.pallas_call(
        body,
        grid=grid,
        in_specs=[
            pl.BlockSpec((1, Mb, 3), lambda b, i: (b, i, 0)),
            pl.BlockSpec((1, 3, N), lambda b, i: (b, 0, 0)),
        ],
        out_specs=pl.BlockSpec((1, Mb, K), lambda b, i: (b, i, 0)),
        out_shape=jax.ShapeDtypeStruct((B, M, K), jnp.int32),
    )


def _knn_idx(pts_src, pts_query, K):
    B, N, _ = pts_src.shape
    M = pts_query.shape[1]
    Mb = min(M, 128 if N >= 8192 else 256)
    pts_t = pts_src.transpose(0, 2, 1)
    return _make_knn(B, N, M, K, Mb)(pts_query, pts_t)


def _gather(arr, idx):
    return jax.vmap(lambda a, i: a[i])(arr, idx)


_SC_CHUNK = 128


@functools.lru_cache(maxsize=None)
def _make_sc_gather(G, R):
    n_w = 32
    per_w = G // n_w
    n_iter = per_w // _SC_CHUNK
    mesh = plsc.VectorSubcoreMesh(core_axis_name="c", subcore_axis_name="s")

    @functools.partial(
        pl.kernel,
        mesh=mesh,
        out_type=jax.ShapeDtypeStruct((G, R), jnp.float32),
        scratch_types=[
            pltpu.VMEM((_SC_CHUNK,), jnp.int32),
            pltpu.VMEM((_SC_CHUNK, R), jnp.float32),
            pltpu.SemaphoreType.DMA,
        ],
    )
    def k(table_hbm, gid_hbm, out_hbm, idx_v, rows_v, sem):
        wid = lax.axis_index("s") * 2 + lax.axis_index("c")
        base = wid * per_w

        def one(i, carry):
            off = base + i * _SC_CHUNK
            pltpu.sync_copy(gid_hbm.at[pl.ds(off, _SC_CHUNK)], idx_v)
            pltpu.async_copy(table_hbm.at[idx_v], rows_v, sem).wait()
            pltpu.sync_copy(rows_v, out_hbm.at[pl.ds(off, _SC_CHUNK)])
            return carry

        lax.fori_loop(0, n_iter, one, 0)

    return k


def _sc_gather_pair(x, pts, idx):
    B, N, Cin = x.shape
    M, K = idx.shape[1], idx.shape[2]
    R = ((Cin + 3 + 127) // 128) * 128
    table = jnp.concatenate(
        [x.reshape(B * N, Cin), pts.reshape(B * N, 3),
         jnp.zeros((B * N, R - Cin - 3), jnp.float32)], axis=1)
    gid = (idx + (jnp.arange(B, dtype=jnp.int32) * N)[:, None, None]).reshape(-1)
    rows = _make_sc_gather(B * M * K, R)(table, gid)
    feats = rows[:, :Cin].reshape(B, M, K, Cin)
    ptsg = rows[:, Cin:Cin + 3].reshape(B, M, K, 3)
    return feats, ptsg


def _ptconv(p, x, pts, K, next_pts, use_sc=True):
    B, N, Cin = x.shape
    if isinstance(next_pts, int):
        if next_pts == N:
            npts = pts
        else:
            stride = N // next_pts
            npts = pts[:, ::stride][:, :next_pts]
    else:
        npts = next_pts
    M = npts.shape[1]
    idx = _knn_idx(pts, npts, K)
    if use_sc and (B * M * K) % (32 * _SC_CHUNK) == 0:
        feats, ptsg = _sc_gather_pair(x, pts, idx)
    else:
        feats = _gather(x, idx)
        ptsg = _gather(pts, idx)
    rel = ptsg - npts[:, :, None, :]
    maxi = jnp.sqrt(jnp.max(jnp.sum(jax.lax.stop_gradient(rel) ** 2, axis=-1), axis=-1))
    maxi = jnp.where(maxi == 0.0, 1.0, maxi)
    rel = rel / maxi[:, :, None, None]
    dists = (rel[:, :, :, None, :] - p["centers"]).reshape(B, M, K, -1)
    h = jax.nn.relu(dists @ p["l1w"] + p["l1b"])
    h = jax.nn.relu(h @ p["l2w"] + p["l2b"])
    h = jax.nn.relu(h @ p["l3w"] + p["l3b"])
    fs = jnp.einsum('bmkc,bmkn->bmcn', feats, h).reshape(B, M, -1)
    out = fs @ p["weight"].reshape(-1, p["weight"].shape[2])
    out = out / K
    return out, npts


def _bn(bp, x):
    mean = jnp.mean(x, axis=(0, 1))
    var = jnp.var(x, axis=(0, 1))
    return bp["gamma"] * (x - mean) / jnp.sqrt(var + 1e-5) + bp["beta"]


def _fc_kernel(x_ref, w_ref, b_ref, o_ref):
    o_ref[...] = jnp.dot(x_ref[...], w_ref[...],
                         preferred_element_type=jnp.float32) + b_ref[...]


def _fc_pallas(x2d, w, b):
    R, C = x2d.shape
    O = w.shape[1]
    return pl.pallas_call(
        _fc_kernel,
        out_shape=jax.ShapeDtypeStruct((R, O), jnp.float32),
    )(x2d, w, b[None, :])


def kernel(x, input_pts, params):
    r = jax.nn.relu
    x1, pts1 = _ptconv(params["cv1"], x, input_pts, 16, 2048)
    x1 = r(_bn(params["bn_cv1"], x1))
    x2, pts2 = _ptconv(params["cv2"], x1, pts1, 16, 1024)
    x2 = r(_bn(params["bn_cv2"], x2))
    x3, pts3 = _ptconv(params["cv3"], x2, pts2, 16, 256)
    x3 = r(_bn(params["bn_cv3"], x3))
    x4, pts4 = _ptconv(params["cv4"], x3, pts3, 8, 64)
    x4 = r(_bn(params["bn_cv4"], x4))
    x5, pts5 = _ptconv(params["cv5"], x4, pts4, 8, 16)
    x5 = r(_bn(params["bn_cv5"], x5))
    x6, pts6 = _ptconv(params["cv6"], x5, pts5, 4, 8)
    x6 = r(_bn(params["bn_cv6"], x6))
    x5d, _ = _ptconv(params["cv5d"], x6, pts6, 4, pts5)
    x5d = r(_bn(params["bn_cv5d"], x5d))
    x5d = jnp.concatenate([x5d, x5], axis=2)
    x4d, _ = _ptconv(params["cv4d"], x5d, pts5, 4, pts4)
    x4d = r(_bn(params["bn_cv4d"], x4d))
    x4d = jnp.concatenate([x4d, x4], axis=2)
    x3d, _ = _ptconv(params["cv3d"], x4d, pts4, 4, pts3)
    x3d = r(_bn(params["bn_cv3d"], x3d))
    x3d = jnp.concatenate([x3d, x3], axis=2)
    x2d, _ = _ptconv(params["cv2d"], x3d, pts3, 8, pts2)
    x2d = r(_bn(params["bn_cv2d"], x2d))
    x2d = jnp.concatenate([x2d, x2], axis=2)
    x1d, _ = _ptconv(params["cv1d"], x2d, pts2, 8, pts1)
    x1d = r(_bn(params["bn_cv1d"], x1d))
    x1d = jnp.concatenate([x1d, x1], axis=2)
    x0d, _ = _ptconv(params["cv0d"], x1d, pts1, 8, input_pts)
    x0d = r(_bn(params["bn_cv0d"], x0d))
    B, M, C = x0d.shape
    xout = _fc_pallas(x0d.reshape(-1, C), params["fcout_w"], params["fcout_b"])
    return xout.reshape(B, -1, xout.shape[1])
